# baseline (device time: 37493 ns/iter reference)
import jax
import jax.numpy as jnp
from jax import lax
from jax.experimental import pallas as pl
from jax.experimental.pallas import tpu as pltpu

N_DEV = 4
B = 2
SQ = 128
SKV_SHARD = 128
D = 512
HQ = 8
HKV = 2
DH = 64
GROUP = HQ // HKV


def kernel(x, Wq, Wo, K_ext, V_ext):
    x2d = x.reshape(B * SQ, D)

    def body(x_ref, wq_ref, wo_ref, k_ref, v_ref, out_ref,
             kfull, vfull, kcomm, vcomm, ksend, krecv, vsend, vrecv,
             attn):
        my = lax.axis_index("i")
        left = (my - 1) % N_DEV
        right = (my + 1) % N_DEV

        barrier_sem = pltpu.get_barrier_semaphore()
        for nbr in (left, right):
            pl.semaphore_signal(
                barrier_sem, inc=1,
                device_id=(nbr,), device_id_type=pl.DeviceIdType.MESH,
            )
        pl.semaphore_wait(barrier_sem, 2)

        kfull[:, pl.ds(my * SKV_SHARD, SKV_SHARD), :, :] = k_ref[...]
        vfull[:, pl.ds(my * SKV_SHARD, SKV_SHARD), :, :] = v_ref[...]
        kcomm[0] = k_ref[...]
        vcomm[0] = v_ref[...]

        for h in range(N_DEV - 1):
            rk = pltpu.make_async_remote_copy(
                src_ref=kcomm.at[h],
                dst_ref=kcomm.at[h + 1],
                send_sem=ksend.at[h],
                recv_sem=krecv.at[h],
                device_id=(right,),
                device_id_type=pl.DeviceIdType.MESH,
            )
            rv = pltpu.make_async_remote_copy(
                src_ref=vcomm.at[h],
                dst_ref=vcomm.at[h + 1],
                send_sem=vsend.at[h],
                recv_sem=vrecv.at[h],
                device_id=(right,),
                device_id_type=pl.DeviceIdType.MESH,
            )
            rk.start()
            rv.start()
            rk.wait()
            rv.wait()
            origin = (my - h - 1) % N_DEV
            kfull[:, pl.ds(origin * SKV_SHARD, SKV_SHARD), :, :] = kcomm[h + 1]
            vfull[:, pl.ds(origin * SKV_SHARD, SKV_SHARD), :, :] = vcomm[h + 1]

        q2d = jnp.dot(x_ref[...], wq_ref[...],
                      preferred_element_type=jnp.float32)
        for b in range(B):
            for g in range(HKV):
                kg = kfull[b, :, g, :]
                vg = vfull[b, :, g, :]
                for hh in range(GROUP):
                    head = g * GROUP + hh
                    qh = q2d[b * SQ:(b + 1) * SQ, head * DH:(head + 1) * DH]
                    s = lax.dot_general(
                        qh, kg, (((1,), (1,)), ((), ())),
                        preferred_element_type=jnp.float32,
                    ) * 0.125
                    m = jnp.max(s, axis=1, keepdims=True)
                    p = jnp.exp(s - m)
                    l = jnp.sum(p, axis=1, keepdims=True)
                    o = jnp.dot(p, vg, preferred_element_type=jnp.float32) / l
                    attn[b * SQ:(b + 1) * SQ, head * DH:(head + 1) * DH] = o

        out_ref[...] = jnp.dot(attn[...], wo_ref[...],
                               preferred_element_type=jnp.float32)

    out2d = pl.pallas_call(
        body,
        out_shape=jax.ShapeDtypeStruct((B * SQ, D), jnp.float32),
        in_specs=[
            pl.BlockSpec(memory_space=pltpu.VMEM),
            pl.BlockSpec(memory_space=pltpu.VMEM),
            pl.BlockSpec(memory_space=pltpu.VMEM),
            pl.BlockSpec(memory_space=pltpu.VMEM),
            pl.BlockSpec(memory_space=pltpu.VMEM),
        ],
        out_specs=pl.BlockSpec(memory_space=pltpu.VMEM),
        scratch_shapes=[
            pltpu.VMEM((B, N_DEV * SKV_SHARD, HKV, DH), jnp.float32),
            pltpu.VMEM((B, N_DEV * SKV_SHARD, HKV, DH), jnp.float32),
            pltpu.VMEM((N_DEV, B, SKV_SHARD, HKV, DH), jnp.float32),
            pltpu.VMEM((N_DEV, B, SKV_SHARD, HKV, DH), jnp.float32),
            pltpu.SemaphoreType.DMA((N_DEV - 1,)),
            pltpu.SemaphoreType.DMA((N_DEV - 1,)),
            pltpu.SemaphoreType.DMA((N_DEV - 1,)),
            pltpu.SemaphoreType.DMA((N_DEV - 1,)),
            pltpu.VMEM((B * SQ, HQ * DH), jnp.float32),
        ],
        compiler_params=pltpu.CompilerParams(collective_id=0),
    )(x2d, Wq, Wo, K_ext, V_ext)
    return out2d.reshape(B, SQ, D)


# device time: 19069 ns/iter; 1.9662x vs baseline; 1.9662x over previous
import jax
import jax.numpy as jnp
from jax import lax
from jax.experimental import pallas as pl
from jax.experimental.pallas import tpu as pltpu

N_DEV = 4
B = 2
SQ = 128
SKV_SHARD = 128
D = 512
HQ = 8
HKV = 2
DH = 64
HPD = HQ // N_DEV
HCOLS = HPD * DH


def kernel(x, Wq, Wo, K_ext, V_ext):
    my_out = lax.axis_index("i")
    x2d = x.reshape(B * SQ, D)
    wq_my = lax.dynamic_slice(Wq, (0, my_out * HCOLS), (D, HCOLS))
    kt = jnp.transpose(K_ext, (2, 0, 1, 3))
    vt = jnp.transpose(V_ext, (2, 0, 1, 3))

    def body(x_ref, wq_ref, wo_ref, k_ref, v_ref, out_ref,
             kbuf, vbuf, qs, s_scr, attn_my, attn_buf,
             ksend, krecv, vsend, vrecv, asend, arecv, locsem):
        my = lax.axis_index("i")
        my_kvh = my // 2

        barrier_sem = pltpu.get_barrier_semaphore()
        for d in range(1, N_DEV):
            pl.semaphore_signal(
                barrier_sem, inc=1,
                device_id=((my + d) % N_DEV,),
                device_id_type=pl.DeviceIdType.MESH,
            )
        pl.semaphore_wait(barrier_sem, N_DEV - 1)

        ck = pltpu.make_async_copy(k_ref.at[my_kvh], kbuf.at[my], locsem.at[0])
        cv = pltpu.make_async_copy(v_ref.at[my_kvh], vbuf.at[my], locsem.at[1])
        ck.start()
        cv.start()

        p1 = []
        for d in range(1, N_DEV):
            tgt = (my + d) % N_DEV
            kvh_t = tgt // 2
            rk = pltpu.make_async_remote_copy(
                src_ref=k_ref.at[kvh_t], dst_ref=kbuf.at[my],
                send_sem=ksend.at[d - 1], recv_sem=krecv.at[d - 1],
                device_id=(tgt,), device_id_type=pl.DeviceIdType.MESH,
            )
            rv = pltpu.make_async_remote_copy(
                src_ref=v_ref.at[kvh_t], dst_ref=vbuf.at[my],
                send_sem=vsend.at[d - 1], recv_sem=vrecv.at[d - 1],
                device_id=(tgt,), device_id_type=pl.DeviceIdType.MESH,
            )
            rk.start()
            rv.start()
            p1.append((rk, rv))

        qmy = jnp.dot(x_ref[...], wq_ref[...],
                      preferred_element_type=jnp.float32) * 0.125
        for b in range(B):
            for hh in range(HPD):
                qs[b, hh * SQ:(hh + 1) * SQ, :] = (
                    qmy[b * SQ:(b + 1) * SQ, hh * DH:(hh + 1) * DH])

        ck.wait()
        cv.wait()
        for d in range(1, N_DEV):
            src_dev = (my - d) % N_DEV
            pltpu.make_async_remote_copy(
                src_ref=k_ref.at[0], dst_ref=kbuf.at[src_dev],
                send_sem=ksend.at[d - 1], recv_sem=krecv.at[d - 1],
                device_id=(src_dev,), device_id_type=pl.DeviceIdType.MESH,
            ).wait_recv()
            pltpu.make_async_remote_copy(
                src_ref=v_ref.at[0], dst_ref=vbuf.at[src_dev],
                send_sem=vsend.at[d - 1], recv_sem=vrecv.at[d - 1],
                device_id=(src_dev,), device_id_type=pl.DeviceIdType.MESH,
            ).wait_recv()

        for b in range(B):
            qb = qs[b]
            for c in range(N_DEV):
                s_scr[:, c * SKV_SHARD:(c + 1) * SKV_SHARD] = lax.dot_general(
                    qb, kbuf[c, b],
                    (((1,), (1,)), ((), ())),
                    preferred_element_type=jnp.float32,
                )
            p = jnp.exp(s_scr[...])
            linv = 1.0 / jnp.sum(p, axis=1, keepdims=True)
            o = jnp.dot(p[:, 0:SKV_SHARD], vbuf[0, b],
                        preferred_element_type=jnp.float32)
            for c in range(1, N_DEV):
                o = o + jnp.dot(p[:, c * SKV_SHARD:(c + 1) * SKV_SHARD],
                                vbuf[c, b], preferred_element_type=jnp.float32)
            o = o * linv
            for hh in range(HPD):
                attn_my[b * SQ:(b + 1) * SQ, hh * DH:(hh + 1) * DH] = (
                    o[hh * SQ:(hh + 1) * SQ, :])

        ca = pltpu.make_async_copy(attn_my, attn_buf.at[my], locsem.at[2])
        ca.start()
        p3 = []
        for d in range(1, N_DEV):
            tgt = (my + d) % N_DEV
            ra = pltpu.make_async_remote_copy(
                src_ref=attn_my, dst_ref=attn_buf.at[my],
                send_sem=asend.at[d - 1], recv_sem=arecv.at[d - 1],
                device_id=(tgt,), device_id_type=pl.DeviceIdType.MESH,
            )
            ra.start()
            p3.append(ra)
        ca.wait()
        for d in range(1, N_DEV):
            src_dev = (my - d) % N_DEV
            pltpu.make_async_remote_copy(
                src_ref=attn_my, dst_ref=attn_buf.at[src_dev],
                send_sem=asend.at[d - 1], recv_sem=arecv.at[d - 1],
                device_id=(src_dev,), device_id_type=pl.DeviceIdType.MESH,
            ).wait_recv()

        acc = jnp.dot(attn_buf[0], wo_ref[0:HCOLS, :],
                      preferred_element_type=jnp.float32)
        for c in range(1, N_DEV):
            acc = acc + jnp.dot(
                attn_buf[c], wo_ref[c * HCOLS:(c + 1) * HCOLS, :],
                preferred_element_type=jnp.float32)
        out_ref[...] = acc

        for rk, rv in p1:
            rk.wait_send()
            rv.wait_send()
        for ra in p3:
            ra.wait_send()

    out2d = pl.pallas_call(
        body,
        out_shape=jax.ShapeDtypeStruct((B * SQ, D), jnp.float32),
        in_specs=[pl.BlockSpec(memory_space=pltpu.VMEM)] * 5,
        out_specs=pl.BlockSpec(memory_space=pltpu.VMEM),
        scratch_shapes=[
            pltpu.VMEM((N_DEV, B, SKV_SHARD, DH), jnp.float32),
            pltpu.VMEM((N_DEV, B, SKV_SHARD, DH), jnp.float32),
            pltpu.VMEM((B, HPD * SQ, DH), jnp.float32),
            pltpu.VMEM((HPD * SQ, N_DEV * SKV_SHARD), jnp.float32),
            pltpu.VMEM((B * SQ, HCOLS), jnp.float32),
            pltpu.VMEM((N_DEV, B * SQ, HCOLS), jnp.float32),
            pltpu.SemaphoreType.DMA((N_DEV - 1,)),
            pltpu.SemaphoreType.DMA((N_DEV - 1,)),
            pltpu.SemaphoreType.DMA((N_DEV - 1,)),
            pltpu.SemaphoreType.DMA((N_DEV - 1,)),
            pltpu.SemaphoreType.DMA((N_DEV - 1,)),
            pltpu.SemaphoreType.DMA((N_DEV - 1,)),
            pltpu.SemaphoreType.DMA((3,)),
        ],
        compiler_params=pltpu.CompilerParams(collective_id=0),
    )(x2d, wq_my, Wo, kt, vt)
    return out2d.reshape(B, SQ, D)
